# baseline (speedup 1.0000x reference)
import functools

import jax
import jax.numpy as jnp
from jax import lax
from jax.experimental import pallas as pl
from jax.experimental.pallas import tpu as pltpu
from jax.experimental.pallas import tpu_sc as plsc

N = 10000
D = 128
E = 320000
NC, NS = 2, 16
NW = NC * NS
EPW = E // NW
CHUNK = 80
NCHUNK = EPW // CHUNK
NT = N // CHUNK
DW = 16

AC = 128
GC = 16
GRP = 5
EPW2 = GRP * GC * AC
EPAD = NW * EPW2 - E
NPADR = 48
NA = N + NPADR


@functools.lru_cache(maxsize=None)
def _sc_kernels():
    mesh = plsc.VectorSubcoreMesh(core_axis_name="c", subcore_axis_name="s")

    @functools.partial(
        pl.kernel, mesh=mesh,
        out_type=(jax.ShapeDtypeStruct((N, DW), jnp.float32),
                  jax.ShapeDtypeStruct((N, DW), jnp.float32)),
        scratch_types=[pltpu.VMEM((NCHUNK, CHUNK), jnp.int32),
                       pltpu.VMEM((CHUNK, DW), jnp.float32),
                       pltpu.VMEM((CHUNK, DW), jnp.float32),
                       pltpu.VMEM_SHARED((N, DW), jnp.float32)],
    )
    def deg_kernel(dst_hbm, out0, out1, idx_v, ones_v, zbuf, acc_sh):
        c = lax.axis_index("c")
        s = lax.axis_index("s")
        wid = c * NS + s

        def _fill(r, _):
            ones_v[r, pl.ds(0, 16)] = jnp.full((16,), 1.0, jnp.float32)
            zbuf[r, pl.ds(0, 16)] = jnp.zeros((16,), jnp.float32)
            return 0
        lax.fori_loop(0, CHUNK, _fill, 0)

        for k in range((NT + NS - 1) // NS):
            t = s + NS * k

            @pl.when(t < NT)
            def _():
                pltpu.sync_copy(zbuf, acc_sh.at[pl.ds(t * CHUNK, CHUNK)])
        pltpu.sync_copy(dst_hbm.at[wid], idx_v)
        plsc.subcore_barrier()

        def _body(i, _):
            pltpu.sync_copy(ones_v, acc_sh.at[idx_v.at[i]], add=True)
            return 0
        lax.fori_loop(0, NCHUNK, _body, 0)
        plsc.subcore_barrier()

        for k in range((NT + NS - 1) // NS):
            t = s + NS * k

            @pl.when((t < NT) & (c == 0))
            def _():
                pltpu.sync_copy(acc_sh.at[pl.ds(t * CHUNK, CHUNK)],
                                out0.at[pl.ds(t * CHUNK, CHUNK)])

            @pl.when((t < NT) & (c == 1))
            def _():
                pltpu.sync_copy(acc_sh.at[pl.ds(t * CHUNK, CHUNK)],
                                out1.at[pl.ds(t * CHUNK, CHUNK)])

    @functools.partial(
        pl.kernel, mesh=mesh,
        out_type=(jax.ShapeDtypeStruct((N, D), jnp.float32),
                  jax.ShapeDtypeStruct((N, D), jnp.float32)),
        scratch_types=[pltpu.VMEM((GC, AC), jnp.int32),
                       pltpu.VMEM((GC, AC), jnp.int32),
                       pltpu.VMEM((AC, D), jnp.float32),
                       pltpu.VMEM((AC, D), jnp.float32),
                       pltpu.VMEM_SHARED((NA, D), jnp.float32),
                       pltpu.SemaphoreType.DMA,
                       pltpu.SemaphoreType.DMA],
    )
    def agg_kernel(src_hbm, dst_hbm, y_hbm, out0, out1,
                   sbuf, dbuf, rows0, rows1, acc_sh, sem0, sem1):
        c = lax.axis_index("c")
        s = lax.axis_index("s")
        wid = c * NS + s

        def _fill(r, _):
            for k in range(D // 16):
                rows0[r, pl.ds(k * 16, 16)] = jnp.zeros((16,), jnp.float32)
            return 0
        lax.fori_loop(0, CHUNK, _fill, 0)

        for k in range((NT + NS - 1) // NS):
            t = s + NS * k

            @pl.when(t < NT)
            def _():
                pltpu.sync_copy(rows0.at[pl.ds(0, CHUNK)],
                                acc_sh.at[pl.ds(t * CHUNK, CHUNK)])
        plsc.subcore_barrier()

        for g in range(GRP):
            pltpu.sync_copy(src_hbm.at[wid, g], sbuf)
            pltpu.sync_copy(dst_hbm.at[wid, g], dbuf)

            def _chunk(j, _):
                pltpu.async_copy(y_hbm.at[sbuf.at[j]], rows1, sem1).wait()
                pltpu.sync_copy(rows1, acc_sh.at[dbuf.at[j]], add=True)
                return 0
            lax.fori_loop(0, GC, _chunk, 0)
        plsc.subcore_barrier()

        for k in range((NT + NS - 1) // NS):
            t = s + NS * k

            @pl.when((t < NT) & (c == 0))
            def _():
                pltpu.sync_copy(acc_sh.at[pl.ds(t * CHUNK, CHUNK)],
                                out0.at[pl.ds(t * CHUNK, CHUNK)])

            @pl.when((t < NT) & (c == 1))
            def _():
                pltpu.sync_copy(acc_sh.at[pl.ds(t * CHUNK, CHUNK)],
                                out1.at[pl.ds(t * CHUNK, CHUNK)])

    return deg_kernel, agg_kernel


BLK = 1000


def _elu(v):
    return jnp.where(v > 0, v, jnp.exp(v) - 1.0)


def _prep0_body(x_ref, w_ref, d0_ref, d1_ref, dis_ref, y_ref):
    deg = 1.0 + d0_ref[:, :1] + d1_ref[:, :1]
    dis = lax.rsqrt(deg)
    xw = jnp.dot(x_ref[...], w_ref[...], preferred_element_type=jnp.float32)
    dis_ref[...] = jnp.broadcast_to(dis, (BLK, D))
    y_ref[...] = dis * xw


def _prep_body(a0_ref, a1_ref, y_ref, dis_ref, h_ref, b_ref, w_ref,
               h_out, y_out):
    dis = dis_ref[...]
    conv = dis * (a0_ref[...] + a1_ref[...] + y_ref[...]) + b_ref[...]
    h = _elu(conv) + h_ref[...]
    h_out[...] = h
    y_out[...] = dis * jnp.dot(h, w_ref[...], preferred_element_type=jnp.float32)


def _final_body(a0_ref, a1_ref, y_ref, dis_ref, h_ref, b_ref,
                x_ref, h1_ref, wl_ref, bl_ref, out_ref):
    conv = dis_ref[...] * (a0_ref[...] + a1_ref[...] + y_ref[...]) + b_ref[...]
    h3 = _elu(conv) + h_ref[...]
    acc = jnp.dot(x_ref[...], wl_ref[0:D, :], preferred_element_type=jnp.float32)
    acc += jnp.dot(h1_ref[...], wl_ref[D:2 * D, :], preferred_element_type=jnp.float32)
    acc += jnp.dot(h_ref[...], wl_ref[2 * D:3 * D, :], preferred_element_type=jnp.float32)
    acc += jnp.dot(h3, wl_ref[3 * D:4 * D, :], preferred_element_type=jnp.float32)
    out_ref[...] = acc + bl_ref[...]


def _row_spec(w):
    return pl.BlockSpec((BLK, w), lambda i: (i, 0))


def _full_spec(h, w):
    return pl.BlockSpec((h, w), lambda i: (0, 0))


_GRID = (N // BLK,)
_f32 = jnp.float32

_prep0_call = pl.pallas_call(
    _prep0_body,
    grid=_GRID,
    in_specs=[_row_spec(D), _full_spec(D, D), _row_spec(DW), _row_spec(DW)],
    out_specs=[_row_spec(D), _row_spec(D)],
    out_shape=[jax.ShapeDtypeStruct((N, D), _f32)] * 2,
)

_prep_call = pl.pallas_call(
    _prep_body,
    grid=_GRID,
    in_specs=[_row_spec(D)] * 5 + [_full_spec(1, D), _full_spec(D, D)],
    out_specs=[_row_spec(D), _row_spec(D)],
    out_shape=[jax.ShapeDtypeStruct((N, D), _f32)] * 2,
)

_final_call = pl.pallas_call(
    _final_body,
    grid=_GRID,
    in_specs=[_row_spec(D)] * 5 + [_full_spec(1, D)] + [_row_spec(D)] * 2
             + [_full_spec(4 * D, D), _full_spec(1, D)],
    out_specs=_row_spec(D),
    out_shape=jax.ShapeDtypeStruct((N, D), _f32),
)


def kernel(x, edge_index, W0, b0, W1, b1, W2, b2, W_lin, b_lin):
    deg_kernel, agg_kernel = _sc_kernels()
    ei = edge_index.astype(jnp.int32)
    dst = ei[1].reshape(NW, NCHUNK, CHUNK)
    src_pad = jnp.zeros((EPAD,), jnp.int32)
    dst_pad = N + (jnp.arange(EPAD, dtype=jnp.int32) % NPADR)
    srcp = jnp.concatenate([ei[0], src_pad]).reshape(NW, GRP, GC, AC)
    dstp = jnp.concatenate([ei[1], dst_pad]).reshape(NW, GRP, GC, AC)
    b0r = b0.reshape(1, D)
    b1r = b1.reshape(1, D)
    b2r = b2.reshape(1, D)
    blr = b_lin.reshape(1, D)

    d0, d1 = deg_kernel(dst)
    dis, y1 = _prep0_call(x, W0, d0, d1)
    a0, a1 = agg_kernel(srcp, dstp, y1)
    h1, y2 = _prep_call(a0, a1, y1, dis, x, b0r, W1)
    a0, a1 = agg_kernel(srcp, dstp, y2)
    h2, y3 = _prep_call(a0, a1, y2, dis, h1, b1r, W2)
    a0, a1 = agg_kernel(srcp, dstp, y3)
    out = _final_call(a0, a1, y3, dis, h2, b2r, x, h1, W_lin, blr)
    return out


# speedup vs baseline: 1.0777x; 1.0777x over previous
import functools

import jax
import jax.numpy as jnp
from jax import lax
from jax.experimental import pallas as pl
from jax.experimental.pallas import tpu as pltpu
from jax.experimental.pallas import tpu_sc as plsc

N = 10000
D = 128
E = 320000
NC, NS = 2, 16
NW = NC * NS
EPW = E // NW
CHUNK = 80
NCHUNK = EPW // CHUNK
NT = N // CHUNK
DW = 16

AC = 128
GC = 16
GRP = 5
EPW2 = GRP * GC * AC
PADW = EPW2 - EPW
PPW = 8
NPADR = NW * PPW
NA = N + NPADR


@functools.lru_cache(maxsize=None)
def _sc_kernels():
    mesh = plsc.VectorSubcoreMesh(core_axis_name="c", subcore_axis_name="s")

    @functools.partial(
        pl.kernel, mesh=mesh,
        out_type=(jax.ShapeDtypeStruct((N, DW), jnp.float32),
                  jax.ShapeDtypeStruct((N, DW), jnp.float32)),
        scratch_types=[pltpu.VMEM((NCHUNK, CHUNK), jnp.int32),
                       pltpu.VMEM((CHUNK, DW), jnp.float32),
                       pltpu.VMEM((CHUNK, DW), jnp.float32),
                       pltpu.VMEM_SHARED((N, DW), jnp.float32)],
    )
    def deg_kernel(dst_hbm, out0, out1, idx_v, ones_v, zbuf, acc_sh):
        c = lax.axis_index("c")
        s = lax.axis_index("s")
        wid = c * NS + s

        def _fill(r, _):
            ones_v[r, pl.ds(0, 16)] = jnp.full((16,), 1.0, jnp.float32)
            zbuf[r, pl.ds(0, 16)] = jnp.zeros((16,), jnp.float32)
            return 0
        lax.fori_loop(0, CHUNK, _fill, 0)

        for k in range((NT + NS - 1) // NS):
            t = s + NS * k

            @pl.when(t < NT)
            def _():
                pltpu.sync_copy(zbuf, acc_sh.at[pl.ds(t * CHUNK, CHUNK)])
        pltpu.sync_copy(dst_hbm.at[wid], idx_v)
        plsc.subcore_barrier()

        def _body(i, _):
            pltpu.sync_copy(ones_v, acc_sh.at[idx_v.at[i]], add=True)
            return 0
        lax.fori_loop(0, NCHUNK, _body, 0)
        plsc.subcore_barrier()

        for k in range((NT + NS - 1) // NS):
            t = s + NS * k

            @pl.when((t < NT) & (c == 0))
            def _():
                pltpu.sync_copy(acc_sh.at[pl.ds(t * CHUNK, CHUNK)],
                                out0.at[pl.ds(t * CHUNK, CHUNK)])

            @pl.when((t < NT) & (c == 1))
            def _():
                pltpu.sync_copy(acc_sh.at[pl.ds(t * CHUNK, CHUNK)],
                                out1.at[pl.ds(t * CHUNK, CHUNK)])

    @functools.partial(
        pl.kernel, mesh=mesh,
        out_type=(jax.ShapeDtypeStruct((N, D), jnp.float32),
                  jax.ShapeDtypeStruct((N, D), jnp.float32)),
        scratch_types=[pltpu.VMEM((GC, AC), jnp.int32),
                       pltpu.VMEM((GC, AC), jnp.int32),
                       pltpu.VMEM((AC, D), jnp.float32),
                       pltpu.VMEM((AC, D), jnp.float32),
                       pltpu.VMEM_SHARED((NA, D), jnp.float32),
                       pltpu.SemaphoreType.DMA,
                       pltpu.SemaphoreType.DMA],
    )
    def agg_kernel(src_hbm, dst_hbm, y_hbm, out0, out1,
                   sbuf, dbuf, rows0, rows1, acc_sh, sem0, sem1):
        c = lax.axis_index("c")
        s = lax.axis_index("s")
        wid = c * NS + s

        def _fill(r, _):
            for k in range(D // 16):
                rows0[r, pl.ds(k * 16, 16)] = jnp.zeros((16,), jnp.float32)
            return 0
        lax.fori_loop(0, CHUNK, _fill, 0)

        for k in range((NT + NS - 1) // NS):
            t = s + NS * k

            @pl.when(t < NT)
            def _():
                pltpu.sync_copy(rows0.at[pl.ds(0, CHUNK)],
                                acc_sh.at[pl.ds(t * CHUNK, CHUNK)])
        plsc.subcore_barrier()

        for g in range(GRP):
            pltpu.sync_copy(src_hbm.at[wid, g], sbuf)
            pltpu.sync_copy(dst_hbm.at[wid, g], dbuf)

            def _chunk(j, _):
                pltpu.async_copy(y_hbm.at[sbuf.at[j]], rows1, sem1).wait()
                pltpu.sync_copy(rows1, acc_sh.at[dbuf.at[j]], add=True)
                return 0
            lax.fori_loop(0, GC, _chunk, 0)
        plsc.subcore_barrier()

        for k in range((NT + NS - 1) // NS):
            t = s + NS * k

            @pl.when((t < NT) & (c == 0))
            def _():
                pltpu.sync_copy(acc_sh.at[pl.ds(t * CHUNK, CHUNK)],
                                out0.at[pl.ds(t * CHUNK, CHUNK)])

            @pl.when((t < NT) & (c == 1))
            def _():
                pltpu.sync_copy(acc_sh.at[pl.ds(t * CHUNK, CHUNK)],
                                out1.at[pl.ds(t * CHUNK, CHUNK)])

    return deg_kernel, agg_kernel


BLK = 1000


def _elu(v):
    return jnp.where(v > 0, v, jnp.exp(v) - 1.0)


def _prep0_body(x_ref, w_ref, d0_ref, d1_ref, dis_ref, y_ref):
    deg = 1.0 + d0_ref[:, :1] + d1_ref[:, :1]
    dis = lax.rsqrt(deg)
    xw = jnp.dot(x_ref[...], w_ref[...], preferred_element_type=jnp.float32)
    dis_ref[...] = jnp.broadcast_to(dis, (BLK, D))
    y_ref[...] = dis * xw


def _prep_body(a0_ref, a1_ref, y_ref, dis_ref, h_ref, b_ref, w_ref,
               h_out, y_out):
    dis = dis_ref[...]
    conv = dis * (a0_ref[...] + a1_ref[...] + y_ref[...]) + b_ref[...]
    h = _elu(conv) + h_ref[...]
    h_out[...] = h
    y_out[...] = dis * jnp.dot(h, w_ref[...], preferred_element_type=jnp.float32)


def _final_body(a0_ref, a1_ref, y_ref, dis_ref, h_ref, b_ref,
                x_ref, h1_ref, wl_ref, bl_ref, out_ref):
    conv = dis_ref[...] * (a0_ref[...] + a1_ref[...] + y_ref[...]) + b_ref[...]
    h3 = _elu(conv) + h_ref[...]
    acc = jnp.dot(x_ref[...], wl_ref[0:D, :], preferred_element_type=jnp.float32)
    acc += jnp.dot(h1_ref[...], wl_ref[D:2 * D, :], preferred_element_type=jnp.float32)
    acc += jnp.dot(h_ref[...], wl_ref[2 * D:3 * D, :], preferred_element_type=jnp.float32)
    acc += jnp.dot(h3, wl_ref[3 * D:4 * D, :], preferred_element_type=jnp.float32)
    out_ref[...] = acc + bl_ref[...]


def _row_spec(w):
    return pl.BlockSpec((BLK, w), lambda i: (i, 0))


def _full_spec(h, w):
    return pl.BlockSpec((h, w), lambda i: (0, 0))


_GRID = (N // BLK,)
_f32 = jnp.float32

_prep0_call = pl.pallas_call(
    _prep0_body,
    grid=_GRID,
    in_specs=[_row_spec(D), _full_spec(D, D), _row_spec(DW), _row_spec(DW)],
    out_specs=[_row_spec(D), _row_spec(D)],
    out_shape=[jax.ShapeDtypeStruct((N, D), _f32)] * 2,
)

_prep_call = pl.pallas_call(
    _prep_body,
    grid=_GRID,
    in_specs=[_row_spec(D)] * 5 + [_full_spec(1, D), _full_spec(D, D)],
    out_specs=[_row_spec(D), _row_spec(D)],
    out_shape=[jax.ShapeDtypeStruct((N, D), _f32)] * 2,
)

_final_call = pl.pallas_call(
    _final_body,
    grid=_GRID,
    in_specs=[_row_spec(D)] * 5 + [_full_spec(1, D)] + [_row_spec(D)] * 2
             + [_full_spec(4 * D, D), _full_spec(1, D)],
    out_specs=_row_spec(D),
    out_shape=jax.ShapeDtypeStruct((N, D), _f32),
)


def kernel(x, edge_index, W0, b0, W1, b1, W2, b2, W_lin, b_lin):
    deg_kernel, agg_kernel = _sc_kernels()
    ei = edge_index.astype(jnp.int32)
    dst = ei[1].reshape(NW, NCHUNK, CHUNK)
    src_pad = jnp.zeros((NW, PADW), jnp.int32)
    dst_pad = (N + PPW * jnp.arange(NW, dtype=jnp.int32)[:, None]
               + (jnp.arange(PADW, dtype=jnp.int32) % PPW)[None, :])
    srcp = jnp.concatenate([ei[0].reshape(NW, EPW), src_pad],
                           axis=1).reshape(NW, GRP, GC, AC)
    dstp = jnp.concatenate([ei[1].reshape(NW, EPW), dst_pad],
                           axis=1).reshape(NW, GRP, GC, AC)
    b0r = b0.reshape(1, D)
    b1r = b1.reshape(1, D)
    b2r = b2.reshape(1, D)
    blr = b_lin.reshape(1, D)

    d0, d1 = deg_kernel(dst)
    dis, y1 = _prep0_call(x, W0, d0, d1)
    a0, a1 = agg_kernel(srcp, dstp, y1)
    h1, y2 = _prep_call(a0, a1, y1, dis, x, b0r, W1)
    a0, a1 = agg_kernel(srcp, dstp, y2)
    h2, y3 = _prep_call(a0, a1, y2, dis, h1, b1r, W2)
    a0, a1 = agg_kernel(srcp, dstp, y3)
    out = _final_call(a0, a1, y3, dis, h2, b2r, x, h1, W_lin, blr)
    return out


# speedup vs baseline: 2.2093x; 2.0500x over previous
import functools

import jax
import jax.numpy as jnp
from jax import lax
from jax.experimental import pallas as pl
from jax.experimental.pallas import tpu as pltpu
from jax.experimental.pallas import tpu_sc as plsc

N = 10000
D = 128
E = 320000
NC, NS = 2, 16
NW = NC * NS
EPW = E // NW
CHUNK = 80
NCHUNK = EPW // CHUNK
NT = N // CHUNK
DW = 16

AC = 128
GC = 16
GRP = 5
EPW2 = GRP * GC * AC
PADW = EPW2 - EPW
PPW = 8
NPADR = NW * PPW
NA = N + NPADR
PH0 = 64


@functools.lru_cache(maxsize=None)
def _sc_kernels():
    mesh = plsc.VectorSubcoreMesh(core_axis_name="c", subcore_axis_name="s")

    @functools.partial(
        pl.kernel, mesh=mesh,
        out_type=(jax.ShapeDtypeStruct((N, DW), jnp.float32),
                  jax.ShapeDtypeStruct((N, DW), jnp.float32)),
        scratch_types=[pltpu.VMEM((NCHUNK, CHUNK), jnp.int32),
                       pltpu.VMEM((CHUNK, DW), jnp.float32),
                       pltpu.VMEM((CHUNK, DW), jnp.float32),
                       pltpu.VMEM_SHARED((N, DW), jnp.float32)],
    )
    def deg_kernel(dst_hbm, out0, out1, idx_v, ones_v, zbuf, acc_sh):
        c = lax.axis_index("c")
        s = lax.axis_index("s")
        wid = c * NS + s

        def _fill(r, _):
            ones_v[r, pl.ds(0, 16)] = jnp.full((16,), 1.0, jnp.float32)
            zbuf[r, pl.ds(0, 16)] = jnp.zeros((16,), jnp.float32)
            return 0
        lax.fori_loop(0, CHUNK, _fill, 0)

        for k in range((NT + NS - 1) // NS):
            t = s + NS * k

            @pl.when(t < NT)
            def _():
                pltpu.sync_copy(zbuf, acc_sh.at[pl.ds(t * CHUNK, CHUNK)])
        pltpu.sync_copy(dst_hbm.at[wid], idx_v)
        plsc.subcore_barrier()

        def _body(i, _):
            pltpu.sync_copy(ones_v, acc_sh.at[idx_v.at[i]], add=True)
            return 0
        lax.fori_loop(0, NCHUNK, _body, 0)
        plsc.subcore_barrier()

        for k in range((NT + NS - 1) // NS):
            t = s + NS * k

            @pl.when((t < NT) & (c == 0))
            def _():
                pltpu.sync_copy(acc_sh.at[pl.ds(t * CHUNK, CHUNK)],
                                out0.at[pl.ds(t * CHUNK, CHUNK)])

            @pl.when((t < NT) & (c == 1))
            def _():
                pltpu.sync_copy(acc_sh.at[pl.ds(t * CHUNK, CHUNK)],
                                out1.at[pl.ds(t * CHUNK, CHUNK)])

    @functools.partial(
        pl.kernel, mesh=mesh,
        out_type=(jax.ShapeDtypeStruct((N, D), jnp.float32),
                  jax.ShapeDtypeStruct((N, D), jnp.float32)),
        scratch_types=[pltpu.VMEM((NCHUNK, CHUNK), jnp.int32),
                       pltpu.VMEM((NCHUNK, CHUNK), jnp.int32),
                       pltpu.VMEM((CHUNK, D), jnp.float32),
                       pltpu.VMEM_SHARED((N, D), jnp.float32),
                       pltpu.SemaphoreType.DMA],
    )
    def agg_kernel(src_hbm, dst_hbm, y_hbm, out0, out1,
                   sidx, didx, rows0, acc_sh, sem0):
        c = lax.axis_index("c")
        s = lax.axis_index("s")
        wid = c * NS + s

        def _fill(r, _):
            for k in range(D // 16):
                rows0[r, pl.ds(k * 16, 16)] = jnp.zeros((16,), jnp.float32)
            return 0
        lax.fori_loop(0, CHUNK, _fill, 0)

        for k in range((NT + NS - 1) // NS):
            t = s + NS * k

            @pl.when(t < NT)
            def _():
                pltpu.sync_copy(rows0.at[pl.ds(0, CHUNK)],
                                acc_sh.at[pl.ds(t * CHUNK, CHUNK)])
        plsc.subcore_barrier()

        pltpu.sync_copy(src_hbm.at[wid], sidx)
        pltpu.sync_copy(dst_hbm.at[wid], didx)

        def _chunk(j, _):
            pltpu.async_copy(y_hbm.at[sidx.at[j]], rows0, sem0).wait()
            pltpu.sync_copy(rows0, acc_sh.at[didx.at[j]], add=True)
            return 0
        lax.fori_loop(0, NCHUNK, _chunk, 0)
        plsc.subcore_barrier()

        for k in range((NT + NS - 1) // NS):
            t = s + NS * k

            @pl.when((t < NT) & (c == 0))
            def _():
                pltpu.sync_copy(acc_sh.at[pl.ds(t * CHUNK, CHUNK)],
                                out0.at[pl.ds(t * CHUNK, CHUNK)])

            @pl.when((t < NT) & (c == 1))
            def _():
                pltpu.sync_copy(acc_sh.at[pl.ds(t * CHUNK, CHUNK)],
                                out1.at[pl.ds(t * CHUNK, CHUNK)])

    return deg_kernel, agg_kernel


BLK = 1000


def _elu(v):
    return jnp.where(v > 0, v, jnp.exp(v) - 1.0)


def _mm0_body(x_ref, w_ref, xw_ref):
    xw_ref[...] = jnp.dot(x_ref[...], w_ref[...],
                          preferred_element_type=jnp.float32)


def _prep0_body(xw_ref, d0_ref, d1_ref, dis_ref, y_ref):
    deg = 1.0 + d0_ref[:, :1] + d1_ref[:, :1]
    dis = lax.rsqrt(deg)
    dis_ref[...] = jnp.broadcast_to(dis, (BLK, D))
    y_ref[...] = dis * xw_ref[...]


def _prep_body(a0_ref, a1_ref, y_ref, dis_ref, h_ref, b_ref, w_ref,
               h_out, y_out):
    dis = dis_ref[...]
    conv = dis * (a0_ref[...] + a1_ref[...] + y_ref[...]) + b_ref[...]
    h = _elu(conv) + h_ref[...]
    h_out[...] = h
    y_out[...] = dis * jnp.dot(h, w_ref[...], preferred_element_type=jnp.float32)


def _final_body(a0_ref, a1_ref, y_ref, dis_ref, h_ref, b_ref,
                x_ref, h1_ref, wl_ref, bl_ref, out_ref):
    conv = dis_ref[...] * (a0_ref[...] + a1_ref[...] + y_ref[...]) + b_ref[...]
    h3 = _elu(conv) + h_ref[...]
    acc = jnp.dot(x_ref[...], wl_ref[0:D, :], preferred_element_type=jnp.float32)
    acc += jnp.dot(h1_ref[...], wl_ref[D:2 * D, :], preferred_element_type=jnp.float32)
    acc += jnp.dot(h_ref[...], wl_ref[2 * D:3 * D, :], preferred_element_type=jnp.float32)
    acc += jnp.dot(h3, wl_ref[3 * D:4 * D, :], preferred_element_type=jnp.float32)
    out_ref[...] = acc + bl_ref[...]


def _row_spec(w):
    return pl.BlockSpec((BLK, w), lambda i: (i, 0))


def _full_spec(h, w):
    return pl.BlockSpec((h, w), lambda i: (0, 0))


_GRID = (N // BLK,)
_f32 = jnp.float32

_mm0_call = pl.pallas_call(
    _mm0_body,
    grid=_GRID,
    in_specs=[_row_spec(D), _full_spec(D, D)],
    out_specs=_row_spec(D),
    out_shape=jax.ShapeDtypeStruct((N, D), _f32),
)

_prep0_call = pl.pallas_call(
    _prep0_body,
    grid=_GRID,
    in_specs=[_row_spec(D), _row_spec(DW), _row_spec(DW)],
    out_specs=[_row_spec(D), _row_spec(D)],
    out_shape=[jax.ShapeDtypeStruct((N, D), _f32)] * 2,
)

_prep_call = pl.pallas_call(
    _prep_body,
    grid=_GRID,
    in_specs=[_row_spec(D)] * 5 + [_full_spec(1, D), _full_spec(D, D)],
    out_specs=[_row_spec(D), _row_spec(D)],
    out_shape=[jax.ShapeDtypeStruct((N, D), _f32)] * 2,
)

_final_call = pl.pallas_call(
    _final_body,
    grid=_GRID,
    in_specs=[_row_spec(D)] * 5 + [_full_spec(1, D)] + [_row_spec(D)] * 2
             + [_full_spec(4 * D, D), _full_spec(1, D)],
    out_specs=_row_spec(D),
    out_shape=jax.ShapeDtypeStruct((N, D), _f32),
)


def kernel(x, edge_index, W0, b0, W1, b1, W2, b2, W_lin, b_lin):
    deg_kernel, agg_kernel = _sc_kernels()
    ei = edge_index.astype(jnp.int32)
    dst = ei[1].reshape(NW, NCHUNK, CHUNK)
    srcp = ei[0].reshape(NW, NCHUNK, CHUNK)
    dstp = dst
    b0r = b0.reshape(1, D)
    b1r = b1.reshape(1, D)
    b2r = b2.reshape(1, D)
    blr = b_lin.reshape(1, D)

    xw0 = _mm0_call(x, W0)
    d0, d1 = deg_kernel(dst)
    dis, y1 = _prep0_call(xw0, d0, d1)
    a0, a1 = agg_kernel(srcp, dstp, y1)
    h1, y2 = _prep_call(a0, a1, y1, dis, x, b0r, W1)
    a0, a1 = agg_kernel(srcp, dstp, y2)
    h2, y3 = _prep_call(a0, a1, y2, dis, h1, b1r, W2)
    a0, a1 = agg_kernel(srcp, dstp, y3)
    out = _final_call(a0, a1, y3, dis, h2, b2r, x, h1, W_lin, blr)
    return out


# speedup vs baseline: 2.2102x; 1.0004x over previous
import functools

import jax
import jax.numpy as jnp
from jax import lax
from jax.experimental import pallas as pl
from jax.experimental.pallas import tpu as pltpu
from jax.experimental.pallas import tpu_sc as plsc

N = 10000
D = 128
E = 320000
NC, NS = 2, 16
NW = NC * NS
EPW = E // NW
CHUNK = 80
NCHUNK = EPW // CHUNK
NT = N // CHUNK
DW = 16


@functools.lru_cache(maxsize=None)
def _sc_kernels():
    mesh = plsc.VectorSubcoreMesh(core_axis_name="c", subcore_axis_name="s")

    @functools.partial(
        pl.kernel, mesh=mesh,
        out_type=(jax.ShapeDtypeStruct((N, DW), jnp.float32),
                  jax.ShapeDtypeStruct((N, DW), jnp.float32)),
        scratch_types=[pltpu.VMEM((NCHUNK, CHUNK), jnp.int32),
                       pltpu.VMEM((CHUNK, DW), jnp.float32),
                       pltpu.VMEM((CHUNK, DW), jnp.float32),
                       pltpu.VMEM_SHARED((N, DW), jnp.float32)],
    )
    def deg_kernel(dst_hbm, out0, out1, idx_v, ones_v, zbuf, acc_sh):
        c = lax.axis_index("c")
        s = lax.axis_index("s")
        wid = c * NS + s

        def _fill(r, _):
            ones_v[r, pl.ds(0, 16)] = jnp.full((16,), 1.0, jnp.float32)
            zbuf[r, pl.ds(0, 16)] = jnp.zeros((16,), jnp.float32)
            return 0
        lax.fori_loop(0, CHUNK, _fill, 0)

        for k in range((NT + NS - 1) // NS):
            t = s + NS * k

            @pl.when(t < NT)
            def _():
                pltpu.sync_copy(zbuf, acc_sh.at[pl.ds(t * CHUNK, CHUNK)])
        pltpu.sync_copy(dst_hbm.at[wid], idx_v)
        plsc.subcore_barrier()

        def _body(i, _):
            pltpu.sync_copy(ones_v, acc_sh.at[idx_v.at[i]], add=True)
            return 0
        lax.fori_loop(0, NCHUNK, _body, 0)
        plsc.subcore_barrier()

        for k in range((NT + NS - 1) // NS):
            t = s + NS * k

            @pl.when((t < NT) & (c == 0))
            def _():
                pltpu.sync_copy(acc_sh.at[pl.ds(t * CHUNK, CHUNK)],
                                out0.at[pl.ds(t * CHUNK, CHUNK)])

            @pl.when((t < NT) & (c == 1))
            def _():
                pltpu.sync_copy(acc_sh.at[pl.ds(t * CHUNK, CHUNK)],
                                out1.at[pl.ds(t * CHUNK, CHUNK)])

    @functools.partial(
        pl.kernel, mesh=mesh,
        out_type=(jax.ShapeDtypeStruct((N, D), jnp.float32),
                  jax.ShapeDtypeStruct((N, D), jnp.float32)),
        scratch_types=[pltpu.VMEM((NCHUNK, CHUNK), jnp.int32),
                       pltpu.VMEM((NCHUNK, CHUNK), jnp.int32),
                       pltpu.VMEM((CHUNK, D), jnp.float32),
                       pltpu.VMEM_SHARED((N, D), jnp.float32),
                       pltpu.SemaphoreType.DMA],
    )
    def agg_kernel(src_hbm, dst_hbm, y_hbm, out0, out1,
                   sidx, didx, rows0, acc_sh, sem0):
        c = lax.axis_index("c")
        s = lax.axis_index("s")
        wid = c * NS + s

        def _fill(r, _):
            for k in range(D // 16):
                rows0[r, pl.ds(k * 16, 16)] = jnp.zeros((16,), jnp.float32)
            return 0
        lax.fori_loop(0, CHUNK, _fill, 0)

        for k in range((NT + NS - 1) // NS):
            t = s + NS * k

            @pl.when(t < NT)
            def _():
                pltpu.sync_copy(rows0, acc_sh.at[pl.ds(t * CHUNK, CHUNK)])
        plsc.subcore_barrier()

        pltpu.sync_copy(src_hbm.at[wid], sidx)
        pltpu.sync_copy(dst_hbm.at[wid], didx)

        def _chunk(j, _):
            pltpu.async_copy(y_hbm.at[sidx.at[j]], rows0, sem0).wait()
            pltpu.sync_copy(rows0, acc_sh.at[didx.at[j]], add=True)
            return 0
        lax.fori_loop(0, NCHUNK, _chunk, 0)
        plsc.subcore_barrier()

        for k in range((NT + NS - 1) // NS):
            t = s + NS * k

            @pl.when((t < NT) & (c == 0))
            def _():
                pltpu.sync_copy(acc_sh.at[pl.ds(t * CHUNK, CHUNK)],
                                out0.at[pl.ds(t * CHUNK, CHUNK)])

            @pl.when((t < NT) & (c == 1))
            def _():
                pltpu.sync_copy(acc_sh.at[pl.ds(t * CHUNK, CHUNK)],
                                out1.at[pl.ds(t * CHUNK, CHUNK)])

    return deg_kernel, agg_kernel


BLK = 1000


def _elu(v):
    return jnp.where(v > 0, v, jnp.exp(v) - 1.0)


def _mm0_body(x_ref, w_ref, xw_ref):
    xw_ref[...] = jnp.dot(x_ref[...], w_ref[...],
                          preferred_element_type=jnp.float32)


def _prep0_body(xw_ref, d0_ref, d1_ref, dis_ref, y_ref):
    deg = 1.0 + d0_ref[:, :1] + d1_ref[:, :1]
    dis = lax.rsqrt(deg)
    dis_ref[...] = jnp.broadcast_to(dis, (BLK, D))
    y_ref[...] = dis * xw_ref[...]


def _prep_body(a0_ref, a1_ref, y_ref, dis_ref, h_ref, b_ref, w_ref,
               h_out, y_out):
    dis = dis_ref[...]
    conv = dis * (a0_ref[...] + a1_ref[...] + y_ref[...]) + b_ref[...]
    h = _elu(conv) + h_ref[...]
    h_out[...] = h
    y_out[...] = dis * jnp.dot(h, w_ref[...], preferred_element_type=jnp.float32)


def _final_body(a0_ref, a1_ref, y_ref, dis_ref, h_ref, b_ref,
                x_ref, h1_ref, wl_ref, bl_ref, out_ref):
    conv = dis_ref[...] * (a0_ref[...] + a1_ref[...] + y_ref[...]) + b_ref[...]
    h3 = _elu(conv) + h_ref[...]
    acc = jnp.dot(x_ref[...], wl_ref[0:D, :], preferred_element_type=jnp.float32)
    acc += jnp.dot(h1_ref[...], wl_ref[D:2 * D, :], preferred_element_type=jnp.float32)
    acc += jnp.dot(h_ref[...], wl_ref[2 * D:3 * D, :], preferred_element_type=jnp.float32)
    acc += jnp.dot(h3, wl_ref[3 * D:4 * D, :], preferred_element_type=jnp.float32)
    out_ref[...] = acc + bl_ref[...]


def _row_spec(w):
    return pl.BlockSpec((BLK, w), lambda i: (i, 0))


def _full_spec(h, w):
    return pl.BlockSpec((h, w), lambda i: (0, 0))


_GRID = (N // BLK,)
_f32 = jnp.float32

_mm0_call = pl.pallas_call(
    _mm0_body,
    grid=_GRID,
    in_specs=[_row_spec(D), _full_spec(D, D)],
    out_specs=_row_spec(D),
    out_shape=jax.ShapeDtypeStruct((N, D), _f32),
)

_prep0_call = pl.pallas_call(
    _prep0_body,
    grid=_GRID,
    in_specs=[_row_spec(D), _row_spec(DW), _row_spec(DW)],
    out_specs=[_row_spec(D), _row_spec(D)],
    out_shape=[jax.ShapeDtypeStruct((N, D), _f32)] * 2,
)

_prep_call = pl.pallas_call(
    _prep_body,
    grid=_GRID,
    in_specs=[_row_spec(D)] * 5 + [_full_spec(1, D), _full_spec(D, D)],
    out_specs=[_row_spec(D), _row_spec(D)],
    out_shape=[jax.ShapeDtypeStruct((N, D), _f32)] * 2,
)

_final_call = pl.pallas_call(
    _final_body,
    grid=_GRID,
    in_specs=[_row_spec(D)] * 5 + [_full_spec(1, D)] + [_row_spec(D)] * 2
             + [_full_spec(4 * D, D), _full_spec(1, D)],
    out_specs=_row_spec(D),
    out_shape=jax.ShapeDtypeStruct((N, D), _f32),
)


def kernel(x, edge_index, W0, b0, W1, b1, W2, b2, W_lin, b_lin):
    deg_kernel, agg_kernel = _sc_kernels()
    ei = edge_index.astype(jnp.int32)
    dst = ei[1].reshape(NW, NCHUNK, CHUNK)
    srcp = ei[0].reshape(NW, NCHUNK, CHUNK)
    dstp = dst
    b0r = b0.reshape(1, D)
    b1r = b1.reshape(1, D)
    b2r = b2.reshape(1, D)
    blr = b_lin.reshape(1, D)

    xw0 = _mm0_call(x, W0)
    d0, d1 = deg_kernel(dst)
    dis, y1 = _prep0_call(xw0, d0, d1)
    a0, a1 = agg_kernel(srcp, dstp, y1)
    h1, y2 = _prep_call(a0, a1, y1, dis, x, b0r, W1)
    a0, a1 = agg_kernel(srcp, dstp, y2)
    h2, y3 = _prep_call(a0, a1, y2, dis, h1, b1r, W2)
    a0, a1 = agg_kernel(srcp, dstp, y3)
    out = _final_call(a0, a1, y3, dis, h2, b2r, x, h1, W_lin, blr)
    return out
